# baseline (speedup 1.0000x reference)
import jax
import jax.numpy as jnp
from jax import lax
from jax.experimental import pallas as pl
from jax.experimental.pallas import tpu as pltpu

C = 192
K = 3
H = 24
W = 24
HW = H * W
CKK = C * K * K
OFFSETS = [(i - 1, j - 1) for i in range(K) for j in range(K)]
N_ACTIVE = 4


def _shift2d(x, di, dj):
    s = di * W + dj
    if s > 0:
        y = jnp.concatenate(
            [x[:, s:], jnp.zeros((x.shape[0], s), x.dtype)], axis=1)
    elif s < 0:
        y = jnp.concatenate(
            [jnp.zeros((x.shape[0], -s), x.dtype), x[:, :s]], axis=1)
    else:
        y = x
    if dj != 0:
        col = lax.broadcasted_iota(jnp.int32, x.shape, 1) % W
        valid = (col + dj >= 0) & (col + dj < W)
        y = jnp.where(valid, y, jnp.zeros_like(y))
    return y


def _search_kernel(x_ref, norm_ref, prob_ref, s_ref, u_ref, refu_ref, arg_ref):
    x = x_ref[0]
    parts = [_shift2d(x, di, dj) for (di, dj) in OFFSETS]

    norm_cols = jnp.maximum(norm_ref[0], 1e-12)
    un = jnp.concatenate(
        [(parts[o] / norm_cols).astype(jnp.bfloat16) for o in range(N_ACTIVE)],
        axis=0)

    R = lax.dot_general(un, un, (((0,), (0,)), ((), ())),
                        preferred_element_type=jnp.float32)
    p_iota = lax.broadcasted_iota(jnp.int32, (HW, HW), 0)
    q_iota = lax.broadcasted_iota(jnp.int32, (HW, HW), 1)
    Rz = jnp.where(p_iota == q_iota, jnp.float32(0.0), R)
    mx = jnp.max(Rz, axis=0, keepdims=True)
    eq = Rz == mx
    am = jnp.min(jnp.where(eq, p_iota, jnp.int32(HW)), axis=0, keepdims=True)
    arg_ref[0] = am
    s_ref[0] = jnp.clip(mx, 1e-08, 1.0)

    onehot = (p_iota == am).astype(jnp.float32)
    onehot_bf = onehot.astype(jnp.bfloat16)
    dn = (((1,), (0,)), ((), ()))
    raw_bf = jnp.concatenate(
        [parts[o].astype(jnp.bfloat16) for o in range(len(OFFSETS))], axis=0)
    sg = lax.dot_general(raw_bf, onehot_bf, dn,
                         preferred_element_type=jnp.float32)
    sg_bf = sg.astype(jnp.bfloat16)
    gathered = [sg_bf[o * C:(o + 1) * C] for o in range(len(OFFSETS))]
    refu_ref[0] = (jnp.stack(gathered, axis=1)
                   .reshape(CKK, HW).astype(jnp.float32))

    prob = prob_ref[0]
    u = lax.dot_general(prob, onehot, dn,
                        precision=lax.Precision.HIGHEST,
                        preferred_element_type=jnp.float32)
    u_ref[0] = jnp.clip(u, 1e-08, 1.0)


def kernel(y_hat, y_prob, mask_unfold):
    n, c, h, w = y_hat.shape
    sq = jnp.sum(y_hat * y_hat, axis=1)
    sqp = jnp.pad(sq, ((0, 0), (1, 1), (1, 1)))
    norm2 = (sqp[:, 0:H, 0:W] + sqp[:, 0:H, 1:W + 1] + sqp[:, 0:H, 2:W + 2]
             + sqp[:, 1:H + 1, 0:W])
    norm = jnp.sqrt(norm2).reshape(n, 1, HW)
    x3 = y_hat.reshape(n, c, HW)
    prob = y_prob.reshape(n, 1, HW)

    s3, u3, refu, arg3 = pl.pallas_call(
        _search_kernel,
        grid=(n,),
        in_specs=[
            pl.BlockSpec((1, C, HW), lambda i: (i, 0, 0)),
            pl.BlockSpec((1, 1, HW), lambda i: (i, 0, 0)),
            pl.BlockSpec((1, 1, HW), lambda i: (i, 0, 0)),
        ],
        out_specs=[
            pl.BlockSpec((1, 1, HW), lambda i: (i, 0, 0)),
            pl.BlockSpec((1, 1, HW), lambda i: (i, 0, 0)),
            pl.BlockSpec((1, CKK, HW), lambda i: (i, 0, 0)),
            pl.BlockSpec((1, 1, HW), lambda i: (i, 0, 0)),
        ],
        out_shape=[
            jax.ShapeDtypeStruct((n, 1, HW), jnp.float32),
            jax.ShapeDtypeStruct((n, 1, HW), jnp.float32),
            jax.ShapeDtypeStruct((n, CKK, HW), jnp.float32),
            jax.ShapeDtypeStruct((n, 1, HW), jnp.int32),
        ],
        compiler_params=pltpu.CompilerParams(
            dimension_semantics=("parallel",),
        ),
    )(x3, norm, prob)

    S = s3.reshape(n, 1, h, w)
    U = u3.reshape(n, 1, h, w)
    R_star_arg = arg3.reshape(n, HW)
    return (S, U, refu, R_star_arg)
